# Initial kernel scaffold; baseline (speedup 1.0000x reference)
import jax
import jax.numpy as jnp
from jax.experimental import pallas as pl


def kernel(input_ids, position_table, gamma, beta):
    raise NotImplementedError("write your pallas kernel here")


# speedup vs baseline: 1.8003x; 1.8003x over previous
import jax
import jax.numpy as jnp
from jax.experimental import pallas as pl

B, S, H = 4, 4096, 1024
EPS = 1e-12
BLK = 256


def _ln_body(x_ref, pos_ref, gamma_ref, beta_ref, o_ref):
    x = x_ref[...] + pos_ref[...]
    u = jnp.mean(x, axis=-1, keepdims=True)
    xc = x - u
    v = jnp.mean(xc * xc, axis=-1, keepdims=True)
    inv = jax.lax.rsqrt(v + EPS)
    o_ref[...] = xc * inv * gamma_ref[...] + beta_ref[...]


def kernel(input_ids, position_table, gamma, beta):
    x2 = input_ids.reshape(B * S, H)
    g2 = gamma.reshape(1, H)
    b2 = beta.reshape(1, H)
    n_s = S // BLK
    grid = (B, n_s)
    out = pl.pallas_call(
        _ln_body,
        grid=grid,
        in_specs=[
            pl.BlockSpec((BLK, H), lambda i, j: (i * n_s + j, 0)),
            pl.BlockSpec((BLK, H), lambda i, j: (j, 0)),
            pl.BlockSpec((1, H), lambda i, j: (0, 0)),
            pl.BlockSpec((1, H), lambda i, j: (0, 0)),
        ],
        out_specs=pl.BlockSpec((BLK, H), lambda i, j: (i * n_s + j, 0)),
        out_shape=jax.ShapeDtypeStruct((B * S, H), jnp.float32),
    )(x2, position_table, g2, b2)
    return out.reshape(B, S, H)
